# baseline (speedup 1.0000x reference)
import functools

import jax
import jax.numpy as jnp
from jax import lax
from jax.experimental import pallas as pl
from jax.experimental.pallas import tpu as pltpu
from jax.experimental.pallas import tpu_sc as plsc

_CHUNK = 16
_NBUF = 3


def _emb_call(ids_flat, word_emb, pos_emb, n_batch, seq_len):
    n_rows, = ids_flat.shape
    _, d = word_emb.shape

    info = plsc.get_sparse_core_info()
    nc, ns, lanes = info.num_cores, info.num_subcores, info.num_lanes
    nw = nc * ns
    b_per_w = n_rows // nw
    s_per_w = seq_len // nw
    chunk = _CHUNK
    n_chunks = b_per_w // chunk
    cpb = s_per_w // chunk
    nbuf = _NBUF

    mesh = plsc.VectorSubcoreMesh(core_axis_name="c", subcore_axis_name="s")

    scratch = [
        pltpu.VMEM((b_per_w,), jnp.int32),
        pltpu.VMEM((s_per_w, d), jnp.float32),
    ]
    scratch += [pltpu.VMEM((chunk, d), jnp.float32) for _ in range(nbuf)]
    scratch += [pltpu.SemaphoreType.DMA for _ in range(2 * nbuf + 1)]

    @functools.partial(
        pl.kernel,
        mesh=mesh,
        out_type=jax.ShapeDtypeStruct((n_rows, d), jnp.float32),
        scratch_types=scratch,
    )
    def emb_kernel(ids_hbm, wemb_hbm, pemb_hbm, out_hbm, idx_v, pos_v, *bufs):
        rows = bufs[:nbuf]
        gsem = bufs[nbuf:2 * nbuf]
        osem = bufs[2 * nbuf:3 * nbuf]
        isem = bufs[3 * nbuf]

        wid = lax.axis_index("s") * nc + lax.axis_index("c")
        pos0 = wid * s_per_w

        ih = [
            pltpu.async_copy(
                ids_hbm.at[pl.ds(bb * seq_len + pos0, s_per_w)],
                idx_v.at[pl.ds(bb * s_per_w, s_per_w)],
                isem,
            )
            for bb in range(n_batch)
        ]
        ph = pltpu.async_copy(pemb_hbm.at[pl.ds(pos0, s_per_w)], pos_v, gsem[0])
        for h in ih:
            h.wait()
        ph.wait()

        g_handles = [None] * n_chunks
        o_handles = [None] * n_chunks

        def start(c):
            b = c % nbuf
            g_handles[c] = pltpu.async_copy(
                wemb_hbm.at[idx_v.at[pl.ds(c * chunk, chunk)]], rows[b], gsem[b]
            )

        def process(c):
            b = c % nbuf
            roff = (c % cpb) * chunk
            g_handles[c].wait()

            def body(r, carry):
                for j in range(d // lanes):
                    sl = pl.ds(j * lanes, lanes)
                    plsc.addupdate(rows[b].at[r, sl], pos_v[roff + r, sl])
                return carry

            lax.fori_loop(0, chunk, body, 0)
            flat_base = (c // cpb) * seq_len + pos0 + roff
            o_handles[c] = pltpu.async_copy(
                rows[b], out_hbm.at[pl.ds(flat_base, chunk)], osem[b]
            )

        for c in range(n_chunks):
            if c >= nbuf:
                o_handles[c - nbuf].wait()
            start(c)
            if c >= 1:
                process(c - 1)
        process(n_chunks - 1)
        for c in range(n_chunks - nbuf, n_chunks):
            o_handles[c].wait()

    return emb_kernel(ids_flat, word_emb, pos_emb)


def kernel(input_ids, word_emb, pos_emb):
    b, s = input_ids.shape
    _, d = word_emb.shape
    ids_flat = input_ids.reshape(b * s).astype(jnp.int32)
    out = _emb_call(ids_flat, word_emb, pos_emb, b, s)
    return out.reshape(b, s, d)


# speedup vs baseline: 1.1233x; 1.1233x over previous
import functools

import jax
import jax.numpy as jnp
from jax import lax
from jax.experimental import pallas as pl
from jax.experimental.pallas import tpu as pltpu
from jax.experimental.pallas import tpu_sc as plsc

_PCHUNK = 8
_DEPTH = 3


def _emb_call(ids_flat, word_emb, pos_emb, n_batch, seq_len):
    n_rows, = ids_flat.shape
    _, d = word_emb.shape

    info = plsc.get_sparse_core_info()
    nc, ns, lanes = info.num_cores, info.num_subcores, info.num_lanes
    nw = nc * ns
    b_per_w = n_rows // nw
    s_per_w = seq_len // nw
    pchunk = _PCHUNK
    npc = s_per_w // pchunk
    depth = _DEPTH

    mesh = plsc.VectorSubcoreMesh(core_axis_name="c", subcore_axis_name="s")

    scratch = [pltpu.VMEM((b_per_w,), jnp.int32)]
    scratch += [pltpu.VMEM((pchunk, d), jnp.float32) for _ in range(depth)]
    scratch += [
        pltpu.VMEM((pchunk, d), jnp.float32) for _ in range(depth * n_batch)
    ]
    scratch += [pltpu.SemaphoreType.DMA for _ in range(3 * depth + 1)]

    @functools.partial(
        pl.kernel,
        mesh=mesh,
        out_type=jax.ShapeDtypeStruct((n_rows, d), jnp.float32),
        scratch_types=scratch,
    )
    def emb_kernel(ids_hbm, wemb_hbm, pemb_hbm, out_hbm, idx_v, *bufs):
        pos_r = bufs[:depth]
        rows = [
            bufs[depth + s * n_batch:depth + (s + 1) * n_batch]
            for s in range(depth)
        ]
        sems = bufs[depth + depth * n_batch:]
        psem = sems[:depth]
        gsem = sems[depth:2 * depth]
        osem = sems[2 * depth:3 * depth]
        isem = sems[3 * depth]

        wid = lax.axis_index("s") * nc + lax.axis_index("c")
        pos0 = wid * s_per_w

        ih = [
            pltpu.async_copy(
                ids_hbm.at[pl.ds(bb * seq_len + pos0, s_per_w)],
                idx_v.at[pl.ds(bb * s_per_w, s_per_w)],
                isem,
            )
            for bb in range(n_batch)
        ]
        for h in ih:
            h.wait()

        ph = [None] * npc
        gh = [None] * npc
        oh = [None] * npc

        def start(p):
            s = p % depth
            roff = p * pchunk
            ph[p] = pltpu.async_copy(
                pemb_hbm.at[pl.ds(pos0 + roff, pchunk)], pos_r[s], psem[s]
            )
            gh[p] = [
                pltpu.async_copy(
                    wemb_hbm.at[idx_v.at[pl.ds(bb * s_per_w + roff, pchunk)]],
                    rows[s][bb],
                    gsem[s],
                )
                for bb in range(n_batch)
            ]

        def process(p):
            s = p % depth
            roff = p * pchunk
            ph[p].wait()
            for h in gh[p]:
                h.wait()

            def body(r, carry):
                for j in range(d // lanes):
                    sl = pl.ds(j * lanes, lanes)
                    pv = pos_r[s][r, sl]
                    for bb in range(n_batch):
                        plsc.addupdate(rows[s][bb].at[r, sl], pv)
                return carry

            lax.fori_loop(0, pchunk, body, 0)
            oh[p] = [
                pltpu.async_copy(
                    rows[s][bb],
                    out_hbm.at[pl.ds(bb * seq_len + pos0 + roff, pchunk)],
                    osem[s],
                )
                for bb in range(n_batch)
            ]

        for p in range(npc):
            if p >= depth:
                for h in oh[p - depth]:
                    h.wait()
            start(p)
            if p >= 1:
                process(p - 1)
        process(npc - 1)
        for p in range(npc - depth, npc):
            for h in oh[p]:
                h.wait()

    return emb_kernel(ids_flat, word_emb, pos_emb)


def kernel(input_ids, word_emb, pos_emb):
    b, s = input_ids.shape
    _, d = word_emb.shape
    ids_flat = input_ids.reshape(b * s).astype(jnp.int32)
    out = _emb_call(ids_flat, word_emb, pos_emb, b, s)
    return out.reshape(b, s, d)
